# baseline (speedup 1.0000x reference)
import functools

import jax
import jax.numpy as jnp
from jax import lax
from jax.experimental import pallas as pl
from jax.experimental.pallas import tpu as pltpu, tpu_sc as plsc

LEN_RULES = 100000
PAD_TOKEN = 100000
B = 16384
L = 200
NUM_CORES = 2
NUM_SUBCORES = 16
NW = NUM_CORES * NUM_SUBCORES
ROWS_PER_W = B // NW
CHUNK_ROWS = 64
NCHUNK = ROWS_PER_W // CHUNK_ROWS
GROUPS = CHUNK_ROWS // 16


def _sc_body(rules_hbm, table_hbm, out_hbm, table_v, rules_v, out_v):
    wid = lax.axis_index("s") * NUM_CORES + lax.axis_index("c")
    base_row = wid * ROWS_PER_W

    pltpu.sync_copy(table_hbm, table_v)

    lane = lax.iota(jnp.int32, 16)

    for c in range(NCHUNK):
        pltpu.sync_copy(
            rules_hbm.at[pl.ds(base_row + c * CHUNK_ROWS, CHUNK_ROWS), :],
            rules_v,
        )
        for g in range(GROUPS):
            row_idx = lane + g * 16

            def step(l, den):
                rv = plsc.load_gather(
                    rules_v, [row_idx, jnp.full((16,), 0, jnp.int32) + l])
                logit = plsc.load_gather(table_v, [rv])
                f = 1.0 + jnp.exp(logit)
                f = jnp.where(rv == PAD_TOKEN, 1.0, f)
                return den * f

            den = lax.fori_loop(0, L, step, jnp.ones((16,), jnp.float32),
                                unroll=8)
            res = jnp.clip(1.0 - 1.0 / den, 0.0001, 0.99999)
            out_v[pl.ds(c * CHUNK_ROWS + g * 16, 16)] = res

    pltpu.sync_copy(out_v, out_hbm.at[pl.ds(base_row, ROWS_PER_W)])


@functools.partial(jax.jit, static_argnames=())
def kernel(rules, relation, table):
    del relation
    table_flat = table.reshape(-1)
    mesh = plsc.VectorSubcoreMesh(core_axis_name="c", subcore_axis_name="s")
    out = pl.kernel(
        _sc_body,
        out_type=jax.ShapeDtypeStruct((B,), jnp.float32),
        mesh=mesh,
        scratch_types=[
            pltpu.VMEM((LEN_RULES + 1,), jnp.float32),
            pltpu.VMEM((CHUNK_ROWS, L), jnp.int32),
            pltpu.VMEM((ROWS_PER_W,), jnp.float32),
        ],
        compiler_params=pltpu.CompilerParams(needs_layout_passes=False),
    )(rules, table_flat)
    return out.reshape(B, 1)


# speedup vs baseline: 1.1027x; 1.1027x over previous
import functools

import jax
import jax.numpy as jnp
from jax import lax
from jax.experimental import pallas as pl
from jax.experimental.pallas import tpu as pltpu, tpu_sc as plsc

LEN_RULES = 100000
PAD_TOKEN = 100000
B = 16384
L = 200
NUM_CORES = 2
NUM_SUBCORES = 16
NW = NUM_CORES * NUM_SUBCORES
ROWS_PER_W = B // NW
CHUNK_ROWS = 64
NCHUNK = ROWS_PER_W // CHUNK_ROWS
GROUPS = CHUNK_ROWS // 16
T_PAD = 100096
T_SLICE = T_PAD // NUM_SUBCORES


def _sc_body(rules_hbm, table_hbm, out_hbm, table_f, rules_v, out_v, table_spm):
    cid = lax.axis_index("c")
    sid = lax.axis_index("s")
    wid = sid * NUM_CORES + cid
    base_row = wid * ROWS_PER_W

    t_off = sid * T_SLICE
    pltpu.sync_copy(table_hbm.at[pl.ds(t_off, T_SLICE)],
                    table_f.at[pl.ds(t_off, T_SLICE)])

    def xform(i, _):
        o = t_off + i * 16
        table_f[pl.ds(o, 16)] = 1.0 + jnp.exp(table_f[pl.ds(o, 16)])
        return 0

    lax.fori_loop(0, T_SLICE // 16, xform, 0, unroll=4)
    pltpu.sync_copy(table_f.at[pl.ds(t_off, T_SLICE)],
                    table_spm.at[pl.ds(t_off, T_SLICE)])
    plsc.subcore_barrier()
    pltpu.sync_copy(table_spm, table_f)

    lane = lax.iota(jnp.int32, 16)

    for c in range(NCHUNK):
        pltpu.sync_copy(
            rules_hbm.at[pl.ds(base_row + c * CHUNK_ROWS, CHUNK_ROWS), :],
            rules_v,
        )
        for g in range(GROUPS):
            row_idx = lane + g * 16

            def step(l, den):
                rv = plsc.load_gather(
                    rules_v, [row_idx, jnp.full((16,), 0, jnp.int32) + l])
                f = plsc.load_gather(table_f, [rv])
                f = jnp.where(rv == PAD_TOKEN, 1.0, f)
                return den * f

            den = lax.fori_loop(0, L, step, jnp.ones((16,), jnp.float32),
                                unroll=8)
            res = jnp.clip(1.0 - 1.0 / den, 0.0001, 0.99999)
            out_v[pl.ds(c * CHUNK_ROWS + g * 16, 16)] = res

    pltpu.sync_copy(out_v, out_hbm.at[pl.ds(base_row, ROWS_PER_W)])


@functools.partial(jax.jit, static_argnames=())
def kernel(rules, relation, table):
    del relation
    table_p = jnp.pad(table.reshape(-1), (0, T_PAD - (LEN_RULES + 1)))
    mesh = plsc.VectorSubcoreMesh(core_axis_name="c", subcore_axis_name="s")
    out = pl.kernel(
        _sc_body,
        out_type=jax.ShapeDtypeStruct((B,), jnp.float32),
        mesh=mesh,
        scratch_types=[
            pltpu.VMEM((T_PAD,), jnp.float32),
            pltpu.VMEM((CHUNK_ROWS, L), jnp.int32),
            pltpu.VMEM((ROWS_PER_W,), jnp.float32),
            pltpu.VMEM_SHARED((T_PAD,), jnp.float32),
        ],
        compiler_params=pltpu.CompilerParams(needs_layout_passes=False),
    )(rules, table_p)
    return out.reshape(B, 1)


# speedup vs baseline: 1.2639x; 1.1462x over previous
import functools

import jax
import jax.numpy as jnp
from jax import lax
from jax.experimental import pallas as pl
from jax.experimental.pallas import tpu as pltpu, tpu_sc as plsc

LEN_RULES = 100000
PAD_TOKEN = 100000
B = 16384
L = 200
NUM_CORES = 2
NUM_SUBCORES = 16
NW = NUM_CORES * NUM_SUBCORES
ROWS_PER_W = B // NW
CHUNK_ROWS = 64
NCHUNK = ROWS_PER_W // CHUNK_ROWS
GROUPS = CHUNK_ROWS // 16
T_PAD = 100096
T_SLICE = T_PAD // NUM_SUBCORES


def _sc_body(rules_hbm, table_hbm, out_hbm, table_f, rules_v, out_v, table_spm):
    cid = lax.axis_index("c")
    sid = lax.axis_index("s")
    wid = sid * NUM_CORES + cid
    base_row = wid * ROWS_PER_W

    t_off = sid * T_SLICE
    pltpu.sync_copy(table_hbm.at[pl.ds(t_off, T_SLICE)],
                    table_f.at[pl.ds(t_off, T_SLICE)])

    def xform(i, _):
        o = t_off + i * 16
        table_f[pl.ds(o, 16)] = 1.0 + jnp.exp(table_f[pl.ds(o, 16)])
        return 0

    lax.fori_loop(0, T_SLICE // 16, xform, 0, unroll=4)
    pltpu.sync_copy(table_f.at[pl.ds(t_off, T_SLICE)],
                    table_spm.at[pl.ds(t_off, T_SLICE)])
    plsc.subcore_barrier()
    pltpu.sync_copy(table_spm, table_f)

    lane_off = lax.iota(jnp.int32, 16) * L

    for c in range(NCHUNK):
        pltpu.sync_copy(
            rules_hbm.at[pl.ds((base_row + c * CHUNK_ROWS) * L, CHUNK_ROWS * L)],
            rules_v,
        )
        for g in range(GROUPS):
            row_idx = lane_off + g * 16 * L

            def step(l, den):
                rv = plsc.load_gather(rules_v, [row_idx + l])
                f = plsc.load_gather(table_f, [rv])
                f = jnp.where(rv == PAD_TOKEN, 1.0, f)
                return den * f

            den = lax.fori_loop(0, L, step, jnp.ones((16,), jnp.float32),
                                unroll=8)
            res = jnp.clip(1.0 - 1.0 / den, 0.0001, 0.99999)
            out_v[pl.ds(c * CHUNK_ROWS + g * 16, 16)] = res

    pltpu.sync_copy(out_v, out_hbm.at[pl.ds(base_row, ROWS_PER_W)])


@functools.partial(jax.jit, static_argnames=())
def kernel(rules, relation, table):
    del relation
    table_p = jnp.pad(table.reshape(-1), (0, T_PAD - (LEN_RULES + 1)))
    mesh = plsc.VectorSubcoreMesh(core_axis_name="c", subcore_axis_name="s")
    out = pl.kernel(
        _sc_body,
        out_type=jax.ShapeDtypeStruct((B,), jnp.float32),
        mesh=mesh,
        scratch_types=[
            pltpu.VMEM((T_PAD,), jnp.float32),
            pltpu.VMEM((CHUNK_ROWS * L,), jnp.int32),
            pltpu.VMEM((ROWS_PER_W,), jnp.float32),
            pltpu.VMEM_SHARED((T_PAD,), jnp.float32),
        ],
        compiler_params=pltpu.CompilerParams(needs_layout_passes=False),
    )(rules.reshape(-1), table_p)
    return out.reshape(B, 1)


# speedup vs baseline: 2.2952x; 1.8160x over previous
import functools

import jax
import jax.numpy as jnp
from jax import lax
from jax.experimental import pallas as pl
from jax.experimental.pallas import tpu as pltpu, tpu_sc as plsc

LEN_RULES = 100000
PAD_TOKEN = 100000
B = 16384
L = 200
NUM_CORES = 2
NUM_SUBCORES = 16
NW = NUM_CORES * NUM_SUBCORES
ROWS_PER_W = B // NW
CHUNK_ROWS = 128
NCHUNK = ROWS_PER_W // CHUNK_ROWS
GROUPS = CHUNK_ROWS // 16
T_PAD = 100096
T_SLICE = T_PAD // NUM_SUBCORES


def _sc_body(rules_hbm, table_hbm, out_hbm, table_f, rules_v, out_v, table_spm):
    cid = lax.axis_index("c")
    sid = lax.axis_index("s")
    wid = sid * NUM_CORES + cid
    base_row = wid * ROWS_PER_W

    t_off = sid * T_SLICE
    pltpu.sync_copy(table_hbm.at[pl.ds(t_off, T_SLICE)],
                    table_f.at[pl.ds(t_off, T_SLICE)])

    def xform(i, _):
        o = t_off + i * 16
        table_f[pl.ds(o, 16)] = 1.0 + jnp.exp(table_f[pl.ds(o, 16)])
        return 0

    lax.fori_loop(0, T_SLICE // 16, xform, 0, unroll=4)
    pltpu.sync_copy(table_f.at[pl.ds(t_off, T_SLICE)],
                    table_spm.at[pl.ds(t_off, T_SLICE)])
    plsc.subcore_barrier()
    pltpu.sync_copy(table_spm, table_f)

    for c in range(NCHUNK):
        dens = [jnp.ones((16,), jnp.float32)] * GROUPS
        for h_off, h_len in ((0, 104), (104, 96)):
            pltpu.sync_copy(
                rules_hbm.at[pl.ds(h_off, h_len),
                             pl.ds(base_row + c * CHUNK_ROWS, CHUNK_ROWS)],
                rules_v.at[pl.ds(0, h_len), :],
            )
            for g in range(GROUPS):
                col0 = g * 16

                def step(l, den):
                    rv = rules_v[l, pl.ds(col0, 16)]
                    f = plsc.load_gather(table_f, [rv])
                    f = jnp.where(rv == PAD_TOKEN, 1.0, f)
                    return den * f

                dens[g] = lax.fori_loop(0, h_len, step, dens[g], unroll=8)
        for g in range(GROUPS):
            res = jnp.clip(1.0 - 1.0 / dens[g], 0.0001, 0.99999)
            out_v[pl.ds(c * CHUNK_ROWS + g * 16, 16)] = res

    pltpu.sync_copy(out_v, out_hbm.at[pl.ds(base_row, ROWS_PER_W)])


@functools.partial(jax.jit, static_argnames=())
def kernel(rules, relation, table):
    del relation
    table_p = jnp.pad(table.reshape(-1), (0, T_PAD - (LEN_RULES + 1)))
    mesh = plsc.VectorSubcoreMesh(core_axis_name="c", subcore_axis_name="s")
    out = pl.kernel(
        _sc_body,
        out_type=jax.ShapeDtypeStruct((B,), jnp.float32),
        mesh=mesh,
        scratch_types=[
            pltpu.VMEM((T_PAD,), jnp.float32),
            pltpu.VMEM((104, CHUNK_ROWS), jnp.int32),
            pltpu.VMEM((ROWS_PER_W,), jnp.float32),
            pltpu.VMEM_SHARED((T_PAD,), jnp.float32),
        ],
        compiler_params=pltpu.CompilerParams(needs_layout_passes=False),
    )(rules.T, table_p)
    return out.reshape(B, 1)


# speedup vs baseline: 2.5172x; 1.0967x over previous
import functools

import jax
import jax.numpy as jnp
from jax import lax
from jax.experimental import pallas as pl
from jax.experimental.pallas import tpu as pltpu, tpu_sc as plsc

LEN_RULES = 100000
PAD_TOKEN = 100000
B = 16384
L = 200
NUM_CORES = 2
NUM_SUBCORES = 16
NW = NUM_CORES * NUM_SUBCORES
ROWS_PER_W = B // NW
CHUNK_ROWS = 128
NCHUNK = ROWS_PER_W // CHUNK_ROWS
GROUPS = CHUNK_ROWS // 16
PARTS = ((0, 56), (56, 56), (112, 56), (168, 32))
T_PAD = 100096
T_SLICE = T_PAD // NUM_SUBCORES


def _sc_body(rules_hbm, table_hbm, out_hbm,
             table_f, rules_v0, rules_v1, out_v, table_spm, sem0, sem1):
    cid = lax.axis_index("c")
    sid = lax.axis_index("s")
    wid = sid * NUM_CORES + cid
    base_row = wid * ROWS_PER_W

    bufs = (rules_v0, rules_v1)
    sems = (sem0, sem1)
    steps = [(c, off, ln) for c in range(NCHUNK) for off, ln in PARTS]

    def start(i):
        c, off, ln = steps[i]
        return pltpu.async_copy(
            rules_hbm.at[pl.ds(off, ln),
                         pl.ds(base_row + c * CHUNK_ROWS, CHUNK_ROWS)],
            bufs[i % 2].at[pl.ds(0, ln), :],
            sems[i % 2],
        )

    pending = start(0)

    t_off = sid * T_SLICE
    pltpu.sync_copy(table_hbm.at[pl.ds(t_off, T_SLICE)],
                    table_f.at[pl.ds(t_off, T_SLICE)])

    def xform(i, _):
        o = t_off + i * 16
        table_f[pl.ds(o, 16)] = 1.0 + jnp.exp(table_f[pl.ds(o, 16)])
        return 0

    lax.fori_loop(0, T_SLICE // 16, xform, 0, unroll=8)
    pltpu.sync_copy(table_f.at[pl.ds(t_off, T_SLICE)],
                    table_spm.at[pl.ds(t_off, T_SLICE)])
    plsc.subcore_barrier()
    pltpu.sync_copy(table_spm, table_f)

    dens = None
    for i, (c, off, ln) in enumerate(steps):
        pending.wait()
        if i + 1 < len(steps):
            nxt = start(i + 1)
        rules_v = bufs[i % 2]
        if off == 0:
            dens = [jnp.ones((16,), jnp.float32)] * GROUPS
        for gp in range(GROUPS // 2):
            col0 = gp * 32

            def step(l, dd):
                da, db = dd
                rva = rules_v[l, pl.ds(col0, 16)]
                rvb = rules_v[l, pl.ds(col0 + 16, 16)]
                fa = plsc.load_gather(table_f, [rva])
                fb = plsc.load_gather(table_f, [rvb])
                fa = jnp.where(rva == PAD_TOKEN, 1.0, fa)
                fb = jnp.where(rvb == PAD_TOKEN, 1.0, fb)
                return (da * fa, db * fb)

            dens[2 * gp], dens[2 * gp + 1] = lax.fori_loop(
                0, ln, step, (dens[2 * gp], dens[2 * gp + 1]), unroll=4)
        if off + ln == L:
            for g in range(GROUPS):
                res = jnp.clip(1.0 - 1.0 / dens[g], 0.0001, 0.99999)
                out_v[pl.ds(c * CHUNK_ROWS + g * 16, 16)] = res
        if i + 1 < len(steps):
            pending = nxt

    pltpu.sync_copy(out_v, out_hbm.at[pl.ds(base_row, ROWS_PER_W)])


@functools.partial(jax.jit, static_argnames=())
def kernel(rules, relation, table):
    del relation
    table_p = jnp.pad(table.reshape(-1), (0, T_PAD - (LEN_RULES + 1)))
    mesh = plsc.VectorSubcoreMesh(core_axis_name="c", subcore_axis_name="s")
    out = pl.kernel(
        _sc_body,
        out_type=jax.ShapeDtypeStruct((B,), jnp.float32),
        mesh=mesh,
        scratch_types=[
            pltpu.VMEM((T_PAD,), jnp.float32),
            pltpu.VMEM((56, CHUNK_ROWS), jnp.int32),
            pltpu.VMEM((56, CHUNK_ROWS), jnp.int32),
            pltpu.VMEM((ROWS_PER_W,), jnp.float32),
            pltpu.VMEM_SHARED((T_PAD,), jnp.float32),
            pltpu.SemaphoreType.DMA,
            pltpu.SemaphoreType.DMA,
        ],
        compiler_params=pltpu.CompilerParams(needs_layout_passes=False),
    )(rules.T, table_p)
    return out.reshape(B, 1)
